# Initial kernel scaffold; baseline (speedup 1.0000x reference)
import jax
import jax.numpy as jnp
from jax.experimental import pallas as pl


def kernel(x, l_w1, l_b1, l_w2, l_b2, l_w3, l_b3, mlp_w, mlp_b, mlp_bn_g, mlp_bn_b, conv1_w, bn1_g, bn1_b, conv2_w, bn2_g, bn2_b, conv3_w, bn3_g, bn3_b, p1_bn_g, p1_bn_b, p1_w, p1_b, p2_bn_g, p2_bn_b, p2_w, p2_b):
    raise NotImplementedError("write your pallas kernel here")


# speedup vs baseline: 4.9737x; 4.9737x over previous
import functools

import jax
import jax.numpy as jnp
from jax import lax
from jax.experimental import pallas as pl
from jax.experimental.pallas import tpu as pltpu

F32 = jnp.float32
N = 1024
K = 20
B = 4
NT = 256
EPS = 1e-5


def _dot(a, b, dims):
    return lax.dot_general(a, b, (dims, ((), ())), preferred_element_type=F32)


def _leaky(x, s):
    return jnp.where(x >= 0, x, s * x)


def _topk_iter(dist, colidx):
    m = jnp.max(dist, axis=1, keepdims=True)
    ismax = dist == m
    idxj = jnp.min(jnp.where(ismax, colidx, jnp.float32(2.0**30)), axis=1,
                   keepdims=True)
    oh = (colidx == idxj).astype(F32)
    dist = jnp.where(colidx == idxj, -jnp.inf, dist)
    return oh, dist


def _split3(F):
    hi = F.astype(jnp.bfloat16).astype(F32)
    r = F - hi
    mid = r.astype(jnp.bfloat16).astype(F32)
    lo = r - mid
    return hi, mid, lo


def _gather_exact(parts, oh):
    hi, mid, lo = parts
    g = _dot(hi, oh, ((1,), (1,)))
    g = g + _dot(mid, oh, ((1,), (1,)))
    g = g + _dot(lo, oh, ((1,), (1,)))
    return g


def _xx_seq(F):
    acc = F[0:1] * F[0:1]
    for c in range(1, F.shape[0]):
        acc = acc + F[c:c + 1] * F[c:c + 1]
    return acc


def _dist_rows(Ft, F, eyeT):
    XtX = _dot(Ft, F, ((0,), (0,)))
    xsq_row = _xx_seq(F)
    xsq_t = _xx_seq(Ft)
    xsq_col = _dot(eyeT, xsq_t, ((1,), (1,)))
    return (-xsq_col + 2.0 * XtX) - xsq_row


def _lafe_body(x_ref, w1, b1, w2, b2, w3, b3, mlp_w, mlp_b,
               f1_ref, hm_ref, mstats_ref, ef_ref, lr_ref):
    b = pl.program_id(0)
    X = x_ref[0]
    colidx = lax.broadcasted_iota(jnp.int32, (N, N), 1).astype(F32)
    rowidx = lax.broadcasted_iota(jnp.int32, (N, N), 0).astype(F32)
    eye = (colidx == rowidx).astype(F32)
    dist = _dist_rows(X, X, eye)
    xparts = _split3(X)
    s = _dot(w3[...], _dot(w1[...], X, ((1,), (0,))) + b1[...],
             ((1,), (0,))) + b3[...]
    for j in range(K):
        oh, dist = _topk_iter(dist, colidx)
        Xg = _gather_exact(xparts, oh)
        efj = _dot(w2[...], X - Xg, ((1,), (0,))) + b2[...]
        naj = _dot(w3[...], efj, ((1,), (0,))) + b3[...]
        lr_ref[j, :] = _leaky(s + naj, 0.01)[0]
        ef_ref[j] = efj
    lr = lr_ref[...]
    mx = jnp.max(lr, axis=0, keepdims=True)
    e = jnp.exp(lr - mx)
    w = e / jnp.sum(e, axis=0, keepdims=True)
    vals = jnp.sum(ef_ref[...] * w[:, None, :], axis=0)
    xm = jnp.where(vals > 0, vals, jnp.exp(jnp.minimum(vals, 0.0)) - 1.0)
    f1 = jnp.concatenate([X, xm], axis=0)
    f1_ref[0] = f1
    hm = _dot(mlp_w[...], f1, ((1,), (0,))) + mlp_b[...]
    hm_ref[0] = hm

    @pl.when(b == 0)
    def _():
        mstats_ref[...] = jnp.zeros_like(mstats_ref)

    mstats_ref[:, 0:1] += jnp.sum(hm, axis=1, keepdims=True)
    mstats_ref[:, 1:2] += jnp.sum(hm * hm, axis=1, keepdims=True)


def _lafe_call(x, w1, b1, w2, b2, w3, b3, mlp_w, mlp_b):
    full = lambda *s: pl.BlockSpec(s, lambda b: tuple(0 for _ in s))
    return pl.pallas_call(
        _lafe_body,
        grid=(B,),
        in_specs=[
            pl.BlockSpec((1, 3, N), lambda b: (b, 0, 0)),
            full(16, 3), full(16, 1), full(16, 3), full(16, 1),
            full(1, 16), full(1, 1), full(256, 19), full(256, 1),
        ],
        out_specs=[
            pl.BlockSpec((1, 19, N), lambda b: (b, 0, 0)),
            pl.BlockSpec((1, 256, N), lambda b: (b, 0, 0)),
            pl.BlockSpec((256, 8), lambda b: (0, 0)),
        ],
        out_shape=[
            jax.ShapeDtypeStruct((B, 19, N), F32),
            jax.ShapeDtypeStruct((B, 256, N), F32),
            jax.ShapeDtypeStruct((256, 8), F32),
        ],
        scratch_shapes=[
            pltpu.VMEM((K, 16, N), F32),
            pltpu.VMEM((K, N), F32),
        ],
    )(x, w1, b1, w2, b2, w3, b3, mlp_w, mlp_b)


def _stage_a_body(f_ref, w_ref, pre_ref, stats_ref, *, C):
    b = pl.program_id(0)
    t = pl.program_id(1)
    F = f_ref[0]

    @pl.when((b == 0) & (t == 0))
    def _():
        stats_ref[...] = jnp.zeros_like(stats_ref)

    fparts = _split3(F)
    Ft = f_ref[0, :, pl.ds(t * NT, NT)]
    colidx = lax.broadcasted_iota(jnp.int32, (NT, N), 1).astype(F32)
    eyeT = (lax.broadcasted_iota(jnp.int32, (NT, NT), 1) ==
            lax.broadcasted_iota(jnp.int32, (NT, NT), 0)).astype(F32)
    dist = _dist_rows(Ft, F, eyeT)
    for j in range(K):
        oh, dist = _topk_iter(dist, colidx)
        Fg = _gather_exact(fparts, oh)
        g = jnp.concatenate([Fg, Ft, Ft - Fg], axis=0)
        prej = _dot(w_ref[...], g, ((1,), (0,)))
        pre_ref[0, j] = prej
        stats_ref[:, 0:1] += jnp.sum(prej, axis=1, keepdims=True)
        stats_ref[:, 1:2] += jnp.sum(prej * prej, axis=1, keepdims=True)


def _stage_a_call(Fin, Wfull, C):
    Cin = Fin.shape[1]
    return pl.pallas_call(
        functools.partial(_stage_a_body, C=C),
        grid=(B, N // NT),
        in_specs=[
            pl.BlockSpec((1, Cin, N), lambda b, t: (b, 0, 0)),
            pl.BlockSpec((C, 3 * Cin), lambda b, t: (0, 0)),
        ],
        out_specs=[
            pl.BlockSpec((1, K, C, NT), lambda b, t: (b, 0, 0, t)),
            pl.BlockSpec((C, 8), lambda b, t: (0, 0)),
        ],
        out_shape=[
            jax.ShapeDtypeStruct((B, K, C, N), F32),
            jax.ShapeDtypeStruct((C, 8), F32),
        ],
    )(Fin, Wfull)


def _stage_b_body(pre_ref, st1_ref, g1_ref, be1_ref, pg_ref, pb_ref,
                  pw_ref, pbias_ref, xp_ref, pst_ref, hh_ref, *, C):
    ph = pl.program_id(0)
    b = pl.program_id(1)
    t = pl.program_id(2)
    cnt = jnp.float32(B * N * K)
    mu1 = st1_ref[:, 0:1] / cnt
    var1 = st1_ref[:, 1:2] / cnt - mu1 * mu1
    sc1 = g1_ref[...] * lax.rsqrt(var1 + EPS)
    sh1 = be1_ref[...] - mu1 * sc1
    pre = pre_ref[0]
    h1 = _leaky(pre * sc1 + sh1, 0.2)

    @pl.when(ph == 0)
    def _():
        @pl.when((b == 0) & (t == 0))
        def _():
            pst_ref[...] = jnp.zeros_like(pst_ref)
        s0 = jnp.sum(h1, axis=0)
        s0q = jnp.sum(h1 * h1, axis=0)
        pst_ref[:, 0:1] += jnp.sum(s0, axis=1, keepdims=True)
        pst_ref[:, 1:2] += jnp.sum(s0q, axis=1, keepdims=True)

    @pl.when(ph == 1)
    def _():
        mu2 = pst_ref[:, 0:1] / cnt
        var2 = pst_ref[:, 1:2] / cnt - mu2 * mu2
        sc2 = pg_ref[...] * lax.rsqrt(var2 + EPS)
        sh2 = pb_ref[...] - mu2 * sc2
        hb = jnp.maximum(h1 * sc2 + sh2, 0.0)
        for j in range(K):
            hh_ref[j] = _dot(pw_ref[...], hb[j], ((1,), (0,))) + pbias_ref[...]
        hh = hh_ref[...]
        mx = jnp.max(hh, axis=0, keepdims=True)
        e = jnp.exp(hh - mx)
        sm = e / jnp.sum(e, axis=0, keepdims=True)
        xp_ref[0] = jnp.sum(h1 * sm, axis=0)


def _stage_b_call(pre, st1, g1, be1, pg, pb, pw, pbias, C):
    return pl.pallas_call(
        functools.partial(_stage_b_body, C=C),
        grid=(2, B, N // NT),
        in_specs=[
            pl.BlockSpec((1, K, C, NT), lambda p, b, t: (b, 0, 0, t)),
            pl.BlockSpec((C, 8), lambda p, b, t: (0, 0)),
            pl.BlockSpec((C, 1), lambda p, b, t: (0, 0)),
            pl.BlockSpec((C, 1), lambda p, b, t: (0, 0)),
            pl.BlockSpec((C, 1), lambda p, b, t: (0, 0)),
            pl.BlockSpec((C, 1), lambda p, b, t: (0, 0)),
            pl.BlockSpec((C, C), lambda p, b, t: (0, 0)),
            pl.BlockSpec((C, 1), lambda p, b, t: (0, 0)),
        ],
        out_specs=pl.BlockSpec((1, C, NT), lambda p, b, t: (b, 0, t)),
        out_shape=jax.ShapeDtypeStruct((B, C, N), F32),
        scratch_shapes=[
            pltpu.VMEM((C, 8), F32),
            pltpu.VMEM((K, C, NT), F32),
        ],
    )(pre, st1, g1, be1, pg, pb, pw, pbias)


def _final_body(hm_ref, mst_ref, mg_ref, mb_ref, xp2_ref, w3_ref,
                g3_ref, b3_ref, out_ref, ybuf_ref, st3_ref):
    ph = pl.program_id(0)
    b = pl.program_id(1)
    cnt = jnp.float32(B * N)

    @pl.when(ph == 0)
    def _():
        mu = mst_ref[:, 0:1] / cnt
        var = mst_ref[:, 1:2] / cnt - mu * mu
        scm = mg_ref[...] * lax.rsqrt(var + EPS)
        shm = mb_ref[...] - mu * scm
        xm = jnp.maximum(hm_ref[0] * scm + shm, 0.0)
        xc2 = jnp.concatenate([xm, xp2_ref[0]], axis=0)
        y = _dot(w3_ref[...], xc2, ((1,), (0,)))
        ybuf_ref[b] = y

        @pl.when(b == 0)
        def _():
            st3_ref[...] = jnp.zeros_like(st3_ref)
        st3_ref[:, 0:1] += jnp.sum(y, axis=1, keepdims=True)
        st3_ref[:, 1:2] += jnp.sum(y * y, axis=1, keepdims=True)

    @pl.when(ph == 1)
    def _():
        mu3 = st3_ref[:, 0:1] / cnt
        var3 = st3_ref[:, 1:2] / cnt - mu3 * mu3
        sc3 = g3_ref[...] * lax.rsqrt(var3 + EPS)
        sh3 = b3_ref[...] - mu3 * sc3
        yn = ybuf_ref[b] * sc3 + sh3
        out_ref[0] = _leaky(yn, 0.2)


def _final_call(hm, mst, mg, mb, xp2, w3, g3, b3):
    return pl.pallas_call(
        _final_body,
        grid=(2, B),
        in_specs=[
            pl.BlockSpec((1, 256, N), lambda p, b: (b, 0, 0)),
            pl.BlockSpec((256, 8), lambda p, b: (0, 0)),
            pl.BlockSpec((256, 1), lambda p, b: (0, 0)),
            pl.BlockSpec((256, 1), lambda p, b: (0, 0)),
            pl.BlockSpec((1, 256, N), lambda p, b: (b, 0, 0)),
            pl.BlockSpec((512, 512), lambda p, b: (0, 0)),
            pl.BlockSpec((512, 1), lambda p, b: (0, 0)),
            pl.BlockSpec((512, 1), lambda p, b: (0, 0)),
        ],
        out_specs=pl.BlockSpec((1, 512, N), lambda p, b: (b, 0, 0)),
        out_shape=jax.ShapeDtypeStruct((B, 512, N), F32),
        scratch_shapes=[
            pltpu.VMEM((B, 512, N), F32),
            pltpu.VMEM((512, 8), F32),
        ],
    )(hm, mst, mg, mb, xp2, w3, g3, b3)


def kernel(x, l_w1, l_b1, l_w2, l_b2, l_w3, l_b3, mlp_w, mlp_b, mlp_bn_g,
           mlp_bn_b, conv1_w, bn1_g, bn1_b, conv2_w, bn2_g, bn2_b, conv3_w,
           bn3_g, bn3_b, p1_bn_g, p1_bn_b, p1_w, p1_b, p2_bn_g, p2_bn_b,
           p2_w, p2_b):
    col = lambda v: jnp.reshape(v, (-1, 1))
    f1, hm, mst = _lafe_call(x, l_w1, col(l_b1), l_w2, col(l_b2), l_w3,
                             col(l_b3), mlp_w, col(mlp_b))

    pre1, st1 = _stage_a_call(f1, conv1_w, 64)
    xp1 = _stage_b_call(pre1, st1, col(bn1_g), col(bn1_b), col(p1_bn_g),
                        col(p1_bn_b), p1_w, col(p1_b), 64)

    xc1 = jnp.concatenate([f1, xp1], axis=1)
    pre2, st2 = _stage_a_call(xc1, conv2_w, 256)
    xp2 = _stage_b_call(pre2, st2, col(bn2_g), col(bn2_b), col(p2_bn_g),
                        col(p2_bn_b), p2_w, col(p2_b), 256)

    return _final_call(hm, mst, col(mlp_bn_g), col(mlp_bn_b), xp2, conv3_w,
                       col(bn3_g), col(bn3_b))


# speedup vs baseline: 6.7542x; 1.3580x over previous
import functools

import jax
import jax.numpy as jnp
from jax import lax
from jax.experimental import pallas as pl
from jax.experimental.pallas import tpu as pltpu
from jax.experimental.pallas import tpu_sc as plsc

F32 = jnp.float32
N = 1024
K = 20
B = 4
NT = 256
CP = 128
EPS = 1e-5


def _dot(a, b, dims):
    return lax.dot_general(a, b, (dims, ((), ())), preferred_element_type=F32)


def _leaky(x, s):
    return jnp.where(x >= 0, x, s * x)


def _split3(F):
    hi = F.astype(jnp.bfloat16).astype(F32)
    r = F - hi
    mid = r.astype(jnp.bfloat16).astype(F32)
    lo = r - mid
    return hi, mid, lo


def _gather_exact(parts, oh):
    hi, mid, lo = parts
    g = _dot(hi, oh, ((1,), (1,)))
    g = g + _dot(mid, oh, ((1,), (1,)))
    g = g + _dot(lo, oh, ((1,), (1,)))
    return g


def _xx_seq(F):
    acc = F[0:1] * F[0:1]
    for c in range(1, F.shape[0]):
        acc = acc + F[c:c + 1] * F[c:c + 1]
    return acc


def _dist_rows(Ft, F, eyeT):
    XtX = _dot(Ft, F, ((0,), (0,)))
    xsq_row = _xx_seq(F)
    xsq_t = _xx_seq(Ft)
    xsq_col = _dot(eyeT, xsq_t, ((1,), (1,)))
    return (-xsq_col + 2.0 * XtX) - xsq_row


def _topk_iter(dist, colidx):
    m = jnp.max(dist, axis=1, keepdims=True)
    ismax = dist == m
    idxj = jnp.min(jnp.where(ismax, colidx, jnp.float32(2.0**30)), axis=1,
                   keepdims=True)
    oh = (colidx == idxj).astype(F32)
    dist = jnp.where(colidx == idxj, -jnp.inf, dist)
    return oh, dist


def _lafe_body(x_ref, w1, b1, w2, b2, w3, b3, mlp_w, mlp_b,
               f1_ref, hm_ref, mstats_ref, ef_ref, lr_ref):
    b = pl.program_id(0)
    X = x_ref[0]
    colidx = lax.broadcasted_iota(jnp.int32, (N, N), 1).astype(F32)
    rowidx = lax.broadcasted_iota(jnp.int32, (N, N), 0).astype(F32)
    eye = (colidx == rowidx).astype(F32)
    dist = _dist_rows(X, X, eye)
    xparts = _split3(X)
    s = _dot(w3[...], _dot(w1[...], X, ((1,), (0,))) + b1[...],
             ((1,), (0,))) + b3[...]
    for j in range(K):
        oh, dist = _topk_iter(dist, colidx)
        Xg = _gather_exact(xparts, oh)
        efj = _dot(w2[...], X - Xg, ((1,), (0,))) + b2[...]
        naj = _dot(w3[...], efj, ((1,), (0,))) + b3[...]
        lr_ref[j, :] = _leaky(s + naj, 0.01)[0]
        ef_ref[j] = efj
    lr = lr_ref[...]
    mx = jnp.max(lr, axis=0, keepdims=True)
    e = jnp.exp(lr - mx)
    w = e / jnp.sum(e, axis=0, keepdims=True)
    vals = jnp.sum(ef_ref[...] * w[:, None, :], axis=0)
    xm = jnp.where(vals > 0, vals, jnp.exp(jnp.minimum(vals, 0.0)) - 1.0)
    f1 = jnp.concatenate([X, xm], axis=0)
    f1_ref[0] = f1
    hm = _dot(mlp_w[...], f1, ((1,), (0,))) + mlp_b[...]
    hm_ref[0] = hm

    @pl.when(b == 0)
    def _():
        mstats_ref[...] = jnp.zeros_like(mstats_ref)

    mstats_ref[:, 0:1] += jnp.sum(hm, axis=1, keepdims=True)
    mstats_ref[:, 1:2] += jnp.sum(hm * hm, axis=1, keepdims=True)


def _lafe_call(x, w1, b1, w2, b2, w3, b3, mlp_w, mlp_b):
    full = lambda *s: pl.BlockSpec(s, lambda b: tuple(0 for _ in s))
    return pl.pallas_call(
        _lafe_body,
        grid=(B,),
        in_specs=[
            pl.BlockSpec((1, 3, N), lambda b: (b, 0, 0)),
            full(16, 3), full(16, 1), full(16, 3), full(16, 1),
            full(1, 16), full(1, 1), full(256, 19), full(256, 1),
        ],
        out_specs=[
            pl.BlockSpec((1, 19, N), lambda b: (b, 0, 0)),
            pl.BlockSpec((1, 256, N), lambda b: (b, 0, 0)),
            pl.BlockSpec((256, 8), lambda b: (0, 0)),
        ],
        out_shape=[
            jax.ShapeDtypeStruct((B, 19, N), F32),
            jax.ShapeDtypeStruct((B, 256, N), F32),
            jax.ShapeDtypeStruct((256, 8), F32),
        ],
        scratch_shapes=[
            pltpu.VMEM((K, 16, N), F32),
            pltpu.VMEM((K, N), F32),
        ],
    )(x, w1, b1, w2, b2, w3, b3, mlp_w, mlp_b)


def _topk_idx_body(f_ref, idx_ref):
    b = pl.program_id(0)
    t = pl.program_id(1)
    F = f_ref[0]
    Ft = f_ref[0, :, pl.ds(t * NT, NT)]
    colidx = lax.broadcasted_iota(jnp.int32, (NT, N), 1).astype(F32)
    eyeT = (lax.broadcasted_iota(jnp.int32, (NT, NT), 1) ==
            lax.broadcasted_iota(jnp.int32, (NT, NT), 0)).astype(F32)
    dist = _dist_rows(Ft, F, eyeT)
    for j in range(K):
        m = jnp.max(dist, axis=1, keepdims=True)
        ismax = dist == m
        idxj = jnp.min(jnp.where(ismax, colidx, jnp.float32(2.0**30)),
                       axis=1, keepdims=True)
        dist = jnp.where(colidx == idxj, -jnp.inf, dist)
        row = jnp.transpose(idxj) + jnp.float32(N) * b.astype(F32)
        idx_ref[0, j] = row.astype(jnp.int32)


def _topk_idx_call(Fin):
    Cin = Fin.shape[1]
    return pl.pallas_call(
        _topk_idx_body,
        grid=(B, N // NT),
        in_specs=[pl.BlockSpec((1, Cin, N), lambda b, t: (b, 0, 0))],
        out_specs=pl.BlockSpec((1, K, 1, NT), lambda b, t: (b, 0, 0, t)),
        out_shape=jax.ShapeDtypeStruct((B, K, 1, N), jnp.int32),
    )(Fin)


def _sc_gather_call(table, idxflat, R):
    info = plsc.get_sparse_core_info()
    NW = info.num_cores * info.num_subcores
    r_per_w = R // NW
    CH = 128
    nch = r_per_w // CH
    mesh = plsc.VectorSubcoreMesh(core_axis_name="c", subcore_axis_name="s")

    @functools.partial(
        pl.kernel, mesh=mesh,
        out_type=jax.ShapeDtypeStruct((R, CP), F32),
        scratch_types=[
            pltpu.VMEM((CH,), jnp.int32),
            pltpu.VMEM((CH, CP), F32),
            pltpu.SemaphoreType.DMA,
        ],
    )
    def k(table_hbm, idx_hbm, out_hbm, idx_v, rows_v, sem):
        wid = lax.axis_index("s") * info.num_cores + lax.axis_index("c")
        base = wid * r_per_w

        def body(i, _):
            off = base + i * CH
            pltpu.sync_copy(idx_hbm.at[pl.ds(off, CH)], idx_v)
            pltpu.async_copy(table_hbm.at[idx_v], rows_v, sem).wait()
            pltpu.sync_copy(rows_v, out_hbm.at[pl.ds(off, CH)])
            return 0

        lax.fori_loop(0, nch, body, 0)

    return k(table, idxflat)


def _rowconv_body(ftab_ref, g_ref, wf_ref, wx_ref, wd_ref,
                  pre_ref, stats_ref, *, Cin):
    b = pl.program_id(0)
    t = pl.program_id(1)

    @pl.when((b == 0) & (t == 0))
    def _():
        stats_ref[...] = jnp.zeros_like(stats_ref)

    Frows = ftab_ref[...][:, :Cin]
    for j in range(K):
        Gj = g_ref[0, j][:, :Cin]
        prej = (_dot(Gj, wf_ref[...], ((1,), (1,))) +
                _dot(Frows, wx_ref[...], ((1,), (1,))) +
                _dot(Frows - Gj, wd_ref[...], ((1,), (1,))))
        pre_ref[0, j] = prej
        stats_ref[0:1, :] += jnp.sum(prej, axis=0, keepdims=True)
        stats_ref[1:2, :] += jnp.sum(prej * prej, axis=0, keepdims=True)


def _rowconv_call(Ftab, G, Wf, Wx, Wd, C):
    Cin = Wf.shape[1]
    NB = N // NT
    return pl.pallas_call(
        functools.partial(_rowconv_body, Cin=Cin),
        grid=(B, NB),
        in_specs=[
            pl.BlockSpec((NT, CP), lambda b, t: (b * NB + t, 0)),
            pl.BlockSpec((1, K, NT, CP), lambda b, t: (b, 0, t, 0)),
            pl.BlockSpec((C, Cin), lambda b, t: (0, 0)),
            pl.BlockSpec((C, Cin), lambda b, t: (0, 0)),
            pl.BlockSpec((C, Cin), lambda b, t: (0, 0)),
        ],
        out_specs=[
            pl.BlockSpec((1, K, NT, C), lambda b, t: (b, 0, t, 0)),
            pl.BlockSpec((8, C), lambda b, t: (0, 0)),
        ],
        out_shape=[
            jax.ShapeDtypeStruct((B, K, N, C), F32),
            jax.ShapeDtypeStruct((8, C), F32),
        ],
    )(Ftab, G, Wf, Wx, Wd)


def _stage_b_row_body(pre_ref, st1_ref, g1_ref, be1_ref, pg_ref, pb_ref,
                      pw_ref, pbias_ref, xp_ref, pst_ref, hh_ref, *, C):
    ph = pl.program_id(0)
    b = pl.program_id(1)
    t = pl.program_id(2)
    cnt = jnp.float32(B * N * K)
    mu1 = st1_ref[0:1, :] / cnt
    var1 = st1_ref[1:2, :] / cnt - mu1 * mu1
    sc1 = g1_ref[...] * lax.rsqrt(var1 + EPS)
    sh1 = be1_ref[...] - mu1 * sc1
    pre = pre_ref[0]
    h1 = _leaky(pre * sc1 + sh1, 0.2)

    @pl.when(ph == 0)
    def _():
        @pl.when((b == 0) & (t == 0))
        def _():
            pst_ref[...] = jnp.zeros_like(pst_ref)
        s0 = jnp.sum(h1, axis=0)
        s0q = jnp.sum(h1 * h1, axis=0)
        pst_ref[0:1, :] += jnp.sum(s0, axis=0, keepdims=True)
        pst_ref[1:2, :] += jnp.sum(s0q, axis=0, keepdims=True)

    @pl.when(ph == 1)
    def _():
        mu2 = pst_ref[0:1, :] / cnt
        var2 = pst_ref[1:2, :] / cnt - mu2 * mu2
        sc2 = pg_ref[...] * lax.rsqrt(var2 + EPS)
        sh2 = pb_ref[...] - mu2 * sc2
        hb = jnp.maximum(h1 * sc2 + sh2, 0.0)
        for j in range(K):
            hh_ref[j] = _dot(hb[j], pw_ref[...], ((1,), (1,))) + pbias_ref[...]
        hh = hh_ref[...]
        mx = jnp.max(hh, axis=0, keepdims=True)
        e = jnp.exp(hh - mx)
        sm = e / jnp.sum(e, axis=0, keepdims=True)
        xp_ref[0] = jnp.sum(h1 * sm, axis=0)


def _stage_b_row_call(pre, st1, g1, be1, pg, pb, pw, pbias, C):
    return pl.pallas_call(
        functools.partial(_stage_b_row_body, C=C),
        grid=(2, B, N // NT),
        in_specs=[
            pl.BlockSpec((1, K, NT, C), lambda p, b, t: (b, 0, t, 0)),
            pl.BlockSpec((8, C), lambda p, b, t: (0, 0)),
            pl.BlockSpec((1, C), lambda p, b, t: (0, 0)),
            pl.BlockSpec((1, C), lambda p, b, t: (0, 0)),
            pl.BlockSpec((1, C), lambda p, b, t: (0, 0)),
            pl.BlockSpec((1, C), lambda p, b, t: (0, 0)),
            pl.BlockSpec((C, C), lambda p, b, t: (0, 0)),
            pl.BlockSpec((1, C), lambda p, b, t: (0, 0)),
        ],
        out_specs=pl.BlockSpec((1, NT, C), lambda p, b, t: (b, t, 0)),
        out_shape=jax.ShapeDtypeStruct((B, N, C), F32),
        scratch_shapes=[
            pltpu.VMEM((8, C), F32),
            pltpu.VMEM((K, NT, C), F32),
        ],
    )(pre, st1, g1, be1, pg, pb, pw, pbias)


def _final_body(hm_ref, mst_ref, mg_ref, mb_ref, xp2_ref, w3_ref,
                g3_ref, b3_ref, out_ref, ybuf_ref, st3_ref):
    ph = pl.program_id(0)
    b = pl.program_id(1)
    cnt = jnp.float32(B * N)

    @pl.when(ph == 0)
    def _():
        mu = mst_ref[:, 0:1] / cnt
        var = mst_ref[:, 1:2] / cnt - mu * mu
        scm = mg_ref[...] * lax.rsqrt(var + EPS)
        shm = mb_ref[...] - mu * scm
        xm = jnp.maximum(hm_ref[0] * scm + shm, 0.0)
        xc2 = jnp.concatenate([xm, xp2_ref[0]], axis=0)
        y = _dot(w3_ref[...], xc2, ((1,), (0,)))
        ybuf_ref[b] = y

        @pl.when(b == 0)
        def _():
            st3_ref[...] = jnp.zeros_like(st3_ref)
        st3_ref[:, 0:1] += jnp.sum(y, axis=1, keepdims=True)
        st3_ref[:, 1:2] += jnp.sum(y * y, axis=1, keepdims=True)

    @pl.when(ph == 1)
    def _():
        mu3 = st3_ref[:, 0:1] / cnt
        var3 = st3_ref[:, 1:2] / cnt - mu3 * mu3
        sc3 = g3_ref[...] * lax.rsqrt(var3 + EPS)
        sh3 = b3_ref[...] - mu3 * sc3
        yn = ybuf_ref[b] * sc3 + sh3
        out_ref[0] = _leaky(yn, 0.2)


def _final_call(hm, mst, mg, mb, xp2, w3, g3, b3):
    return pl.pallas_call(
        _final_body,
        grid=(2, B),
        in_specs=[
            pl.BlockSpec((1, 256, N), lambda p, b: (b, 0, 0)),
            pl.BlockSpec((256, 8), lambda p, b: (0, 0)),
            pl.BlockSpec((256, 1), lambda p, b: (0, 0)),
            pl.BlockSpec((256, 1), lambda p, b: (0, 0)),
            pl.BlockSpec((1, 256, N), lambda p, b: (b, 0, 0)),
            pl.BlockSpec((512, 512), lambda p, b: (0, 0)),
            pl.BlockSpec((512, 1), lambda p, b: (0, 0)),
            pl.BlockSpec((512, 1), lambda p, b: (0, 0)),
        ],
        out_specs=pl.BlockSpec((1, 512, N), lambda p, b: (b, 0, 0)),
        out_shape=jax.ShapeDtypeStruct((B, 512, N), F32),
        scratch_shapes=[
            pltpu.VMEM((B, 512, N), F32),
            pltpu.VMEM((512, 8), F32),
        ],
    )(hm, mst, mg, mb, xp2, w3, g3, b3)


def _sc_stage(Fin, convW, pool_g, pool_b, bn_g, bn_b, pw, pbias, C):
    Cin = Fin.shape[1]
    idx = _topk_idx_call(Fin)
    Ftab = jnp.transpose(Fin, (0, 2, 1)).reshape(B * N, Cin)
    Ftab = jnp.pad(Ftab, ((0, 0), (0, CP - Cin)))
    idxflat = idx.reshape(B * K * N)
    G = _sc_gather_call(Ftab, idxflat, B * K * N).reshape(B, K, N, CP)
    Wf = convW[:, 0:Cin]
    Wx = convW[:, Cin:2 * Cin]
    Wd = convW[:, 2 * Cin:3 * Cin]
    pre, st = _rowconv_call(Ftab, G, Wf, Wx, Wd, C)
    row = lambda v: jnp.reshape(v, (1, -1))
    xp_rows = _stage_b_row_call(pre, st, row(bn_g), row(bn_b), row(pool_g),
                                row(pool_b), pw, row(pbias), C)
    return jnp.transpose(xp_rows, (0, 2, 1))


def kernel(x, l_w1, l_b1, l_w2, l_b2, l_w3, l_b3, mlp_w, mlp_b, mlp_bn_g,
           mlp_bn_b, conv1_w, bn1_g, bn1_b, conv2_w, bn2_g, bn2_b, conv3_w,
           bn3_g, bn3_b, p1_bn_g, p1_bn_b, p1_w, p1_b, p2_bn_g, p2_bn_b,
           p2_w, p2_b):
    col = lambda v: jnp.reshape(v, (-1, 1))
    f1, hm, mst = _lafe_call(x, l_w1, col(l_b1), l_w2, col(l_b2), l_w3,
                             col(l_b3), mlp_w, col(mlp_b))
    xp1 = _sc_stage(f1, conv1_w, p1_bn_g, p1_bn_b, bn1_g, bn1_b,
                    p1_w, p1_b, 64)
    xc1 = jnp.concatenate([f1, xp1], axis=1)
    xp2 = _sc_stage(xc1, conv2_w, p2_bn_g, p2_bn_b, bn2_g, bn2_b,
                    p2_w, p2_b, 256)
    return _final_call(hm, mst, col(mlp_bn_g), col(mlp_bn_b), xp2, conv3_w,
                       col(bn3_g), col(bn3_b))
